# baseline (speedup 1.0000x reference)
import functools

import jax
import jax.numpy as jnp
from jax import lax
from jax.experimental import pallas as pl
from jax.experimental.pallas import tpu as pltpu
from jax.experimental.pallas import tpu_sc as plsc

N = 10000
E = 160000
NP = 10240
EP = 163840
DUMMY = 10000

NWORK = 32
EPW = EP // NWORK
CHUNK = 128
NCHUNK = EPW // CHUNK

F32 = jnp.float32


_NBUF = 4


def _sc_gather(table, idx2d, d):
    mesh = plsc.VectorSubcoreMesh(core_axis_name="c", subcore_axis_name="s")

    @functools.partial(
        pl.kernel, mesh=mesh,
        out_type=jax.ShapeDtypeStruct((EP, d), F32),
        scratch_types=[
            pltpu.VMEM((NCHUNK, CHUNK), jnp.int32),
            [pltpu.VMEM((CHUNK, d), F32) for _ in range(_NBUF)],
            pltpu.SemaphoreType.DMA,
            pltpu.SemaphoreType.DMA,
        ],
    )
    def k(table_hbm, idx_hbm, out_hbm, idx_v, bufs, gsem, wsem):
        wid = lax.axis_index("s") * 2 + lax.axis_index("c")
        base = wid * EPW
        pltpu.sync_copy(idx_hbm.at[pl.ds(wid * NCHUNK, NCHUNK)], idx_v)

        def body(j, carry):
            pltpu.async_copy(table_hbm.at[idx_v.at[j]], bufs[0], gsem).wait()
            pltpu.sync_copy(bufs[0],
                            out_hbm.at[pl.ds(base + j * CHUNK, CHUNK)])
            return carry

        lax.fori_loop(0, NCHUNK, body, 0)

    return k(table, idx2d)


def _sc_scatter_add(msg, dst2d, zeros_np, d):
    mesh = plsc.VectorSubcoreMesh(core_axis_name="c", subcore_axis_name="s")
    rpt = NP // 16

    nb = 2

    @functools.partial(
        pl.kernel, mesh=mesh,
        out_type=jax.ShapeDtypeStruct((2, NP, d), F32),
        scratch_types=[
            pltpu.VMEM((NCHUNK, CHUNK), jnp.int32),
            [pltpu.VMEM((CHUNK, d), F32) for _ in range(nb)],
            pltpu.VMEM_SHARED((NP, d), F32),
            pltpu.SemaphoreType.DMA,
            pltpu.SemaphoreType.DMA,
        ],
    )
    def k(msg_hbm, dst_hbm, zeros_hbm, out_hbm, idx_v, bufs, acc_sh,
          msem, ssem):
        cid = lax.axis_index("c")
        sid = lax.axis_index("s")
        wid = sid * 2 + cid
        base = wid * EPW
        pltpu.sync_copy(zeros_hbm.at[pl.ds(sid * rpt, rpt)],
                        acc_sh.at[pl.ds(sid * rpt, rpt)])
        pltpu.sync_copy(dst_hbm.at[pl.ds(wid * NCHUNK, NCHUNK)], idx_v)
        plsc.subcore_barrier()

        def body(j, carry):
            pltpu.sync_copy(msg_hbm.at[pl.ds(base + j * CHUNK, CHUNK)],
                            bufs[0])
            pltpu.sync_copy(bufs[0], acc_sh.at[idx_v.at[j]], add=True)
            return carry

        lax.fori_loop(0, NCHUNK, body, 0)
        plsc.subcore_barrier()
        pltpu.sync_copy(acc_sh.at[pl.ds(sid * rpt, rpt)],
                        out_hbm.at[cid, pl.ds(sid * rpt, rpt)])

    return k(msg, dst2d, zeros_np)


_ET = 1024


def _tc_msg(ea, hsrc, waug, sbr, tbr, obr, cw, dout, ones_col):
    grid = EP // _ET

    def body(ea_ref, hs_ref, w_ref, s_ref, t_ref, o_ref, out_ref):
        eat = ea_ref[...]
        hs = hs_ref[...][:, :cw]
        eab = jnp.dot(eat, s_ref[...], preferred_element_type=F32) + o_ref[...]
        htl = jnp.dot(hs, t_ref[...], preferred_element_type=F32)
        g = eab * htl
        msg = jnp.dot(g, w_ref[...], preferred_element_type=F32)
        lane = lax.broadcasted_iota(jnp.int32, (_ET, 128 - dout), 1)
        tailv = 1.0 if ones_col else 0.0
        tail = jnp.where(lane == 0, tailv, 0.0).astype(F32)
        out_ref[...] = jnp.concatenate([msg, tail], axis=1)

    return pl.pallas_call(
        body,
        grid=(grid,),
        in_specs=[
            pl.BlockSpec((_ET, 21), lambda i: (i, 0)),
            pl.BlockSpec((_ET, 128), lambda i: (i, 0)),
            pl.BlockSpec((22 * cw, dout), lambda i: (0, 0)),
            pl.BlockSpec((21, 22 * cw), lambda i: (0, 0)),
            pl.BlockSpec((cw, 22 * cw), lambda i: (0, 0)),
            pl.BlockSpec((1, 22 * cw), lambda i: (0, 0)),
        ],
        out_specs=pl.BlockSpec((_ET, 128), lambda i: (i, 0)),
        out_shape=jax.ShapeDtypeStruct((EP, 128), F32),
    )(ea, hsrc, waug, sbr, tbr, obr)


_NT = 512


def _tc_combine1(s0, s1, xp, rootp, biasp, lng, lnb):
    grid = NP // _NT

    def body(s0_ref, s1_ref, x_ref, r_ref, b_ref, g_ref, be_ref,
             h_ref, c_ref):
        s = s0_ref[...] + s1_ref[...]
        cnt = s[:, 64:65]
        agg = s[:, :64] / jnp.maximum(cnt, 1.0)
        pre = agg + jnp.dot(x_ref[...][:, :32], r_ref[...],
                            preferred_element_type=F32) + b_ref[...]
        m = jnp.mean(pre, axis=-1, keepdims=True)
        v = jnp.mean((pre - m) ** 2, axis=-1, keepdims=True)
        h = (pre - m) / jnp.sqrt(v + 1e-5) * g_ref[...] + be_ref[...]
        h = jnp.where(h >= 0, h, 0.01 * h)
        h_ref[...] = jnp.concatenate([h, jnp.zeros((_NT, 64), F32)], axis=1)
        c_ref[...] = jnp.broadcast_to(cnt, (_NT, 8))

    return pl.pallas_call(
        body,
        grid=(grid,),
        in_specs=[
            pl.BlockSpec((_NT, 128), lambda i: (i, 0)),
            pl.BlockSpec((_NT, 128), lambda i: (i, 0)),
            pl.BlockSpec((_NT, 128), lambda i: (i, 0)),
            pl.BlockSpec((32, 64), lambda i: (0, 0)),
            pl.BlockSpec((1, 64), lambda i: (0, 0)),
            pl.BlockSpec((1, 64), lambda i: (0, 0)),
            pl.BlockSpec((1, 64), lambda i: (0, 0)),
        ],
        out_specs=[
            pl.BlockSpec((_NT, 128), lambda i: (i, 0)),
            pl.BlockSpec((_NT, 8), lambda i: (i, 0)),
        ],
        out_shape=[
            jax.ShapeDtypeStruct((NP, 128), F32),
            jax.ShapeDtypeStruct((NP, 8), F32),
        ],
    )(s0, s1, xp, rootp, biasp, lng, lnb)


def _tc_combine2(s0, s1, cnt8, h1s, root2p, bias2p, g2p, b2p):
    grid = NP // _NT

    def body(s0_ref, s1_ref, c_ref, h1_ref, r2_ref, b2_ref, g2_ref, be2_ref,
             h2_ref):
        s = s0_ref[...] + s1_ref[...]
        cnt = c_ref[...][:, 0:1]
        agg = s / jnp.maximum(cnt, 1.0)
        pre = agg + jnp.dot(h1_ref[...], r2_ref[...],
                            preferred_element_type=F32) + b2_ref[...]
        m = jnp.sum(pre, axis=-1, keepdims=True) * (1.0 / 30.0)
        d = pre[:, :30] - m
        v = jnp.sum(d * d, axis=-1, keepdims=True) * (1.0 / 30.0)
        h2 = (pre - m) / jnp.sqrt(v + 1e-5) * g2_ref[...] + be2_ref[...]
        h2_ref[...] = jnp.where(h2 >= 0, h2, 0.01 * h2)

    return pl.pallas_call(
        body,
        grid=(grid,),
        in_specs=[
            pl.BlockSpec((_NT, 32), lambda i: (i, 0)),
            pl.BlockSpec((_NT, 32), lambda i: (i, 0)),
            pl.BlockSpec((_NT, 8), lambda i: (i, 0)),
            pl.BlockSpec((_NT, 64), lambda i: (i, 0)),
            pl.BlockSpec((64, 32), lambda i: (0, 0)),
            pl.BlockSpec((1, 32), lambda i: (0, 0)),
            pl.BlockSpec((1, 32), lambda i: (0, 0)),
            pl.BlockSpec((1, 32), lambda i: (0, 0)),
        ],
        out_specs=pl.BlockSpec((_NT, 32), lambda i: (i, 0)),
        out_shape=jax.ShapeDtypeStruct((NP, 32), F32),
    )(s0, s1, cnt8, h1s, root2p, bias2p, g2p, b2p)


def _tc_lstm_head(h2, wihft, wihbt, xbf, xbb, wcat, g3, b3, fcwt, fcbp, prev):
    ntiles = NP // _NT

    def body(h2_ref, wf_ref, wb_ref, xbf_ref, xbb_ref, wc_ref, g3_ref,
             b3_ref, fw_ref, fb_ref, pr_ref, out_ref, xsum_scr, h_scr):
        def stage1b(j, carry):
            src = h2_ref[pl.ds(j * _NT, _NT), :]
            xb = jnp.dot(src, wb_ref[...],
                         preferred_element_type=F32) + xbb_ref[...]
            rev = jnp.dot(pr_ref[...], xb, preferred_element_type=F32,
                          precision=lax.Precision.HIGHEST)
            xsum_scr[pl.ds((N - 512) - _NT * j, _NT), :] = rev
            return carry

        lax.fori_loop(0, ntiles - 1, stage1b, 0)
        srcl = h2_ref[pl.ds((ntiles - 1) * _NT, _NT), :]
        xbl = jnp.dot(srcl, wb_ref[...],
                      preferred_element_type=F32) + xbb_ref[...]
        revl = jnp.dot(pr_ref[...], xbl, preferred_element_type=F32,
                       precision=lax.Precision.HIGHEST)
        xsum_scr[pl.ds(0, N - (ntiles - 1) * _NT), :] = (
            revl[_NT - (N - (ntiles - 1) * _NT):, :])

        def stage1f(i, carry):
            sl = pl.ds(i * _NT, _NT)
            xsum_scr[sl, :] = (xsum_scr[sl, :]
                               + jnp.dot(h2_ref[sl, :], wf_ref[...],
                                         preferred_element_type=F32)
                               + xbf_ref[...])
            return carry

        lax.fori_loop(0, ntiles, stage1f, 0)

        wc = wc_ref[...]

        def step(t, carry):
            h, c = carry
            rt = (N - 1) - t
            g = jnp.dot(h, wc, preferred_element_type=F32)
            gate = g + xsum_scr[pl.ds(t, 1), :]
            i_a = jax.nn.sigmoid(gate[:, 0:128])
            f_a = jax.nn.sigmoid(gate[:, 128:256])
            n_a = jnp.tanh(gate[:, 256:384])
            o_a = jax.nn.sigmoid(gate[:, 384:512])
            c = f_a * c + i_a * n_a
            h = o_a * jnp.tanh(c)
            h_scr[pl.ds(t, 1), 0:64] = h[:, 0:64]
            h_scr[pl.ds(rt, 1), 64:128] = h[:, 64:128]
            return h, c

        z = jnp.zeros((1, 128), F32)
        lax.fori_loop(0, N, step, (z, z), unroll=2)

        def head(i, carry):
            sl = pl.ds(i * _NT, _NT)
            hc = h_scr[sl, :]
            m = jnp.mean(hc, axis=-1, keepdims=True)
            v = jnp.mean((hc - m) ** 2, axis=-1, keepdims=True)
            hn = (hc - m) / jnp.sqrt(v + 1e-5) * g3_ref[...] + b3_ref[...]
            out_ref[sl, :] = jnp.dot(hn, fw_ref[...],
                                     preferred_element_type=F32) + fb_ref[...]
            return carry

        lax.fori_loop(0, ntiles, head, 0)

    return pl.pallas_call(
        body,
        out_shape=jax.ShapeDtypeStruct((NP, 8), F32),
        scratch_shapes=[
            pltpu.VMEM((NP, 512), F32),
            pltpu.VMEM((NP, 128), F32),
        ],
    )(h2, wihft, wihbt, xbf, xbb, wcat, g3, b3, fcwt, fcbp, prev)


def kernel(x, edge_attr, edge_index, W1e, b1e, root1, bias1, ln1_g, ln1_b,
           W2e, b2e, root2, bias2, ln2_g, ln2_b,
           Wih_f, Whh_f, bih_f, bhh_f, Wih_b, Whh_b, bih_b, bhh_b,
           ln3_g, ln3_b, fcW, fcb):
    src = edge_index[0]
    dst = edge_index[1]
    src_p = jnp.concatenate([src, jnp.zeros((EP - E,), jnp.int32)])
    dst_p = jnp.concatenate([dst, jnp.full((EP - E,), DUMMY, jnp.int32)])
    src2d = src_p.reshape(EP // CHUNK, CHUNK)
    dst2d = dst_p.reshape(EP // CHUNK, CHUNK)
    ea_p = jnp.concatenate([edge_attr, jnp.zeros((EP - E, 21), F32)], axis=0)
    x_p = jnp.zeros((NP, 128), F32).at[:N, :26].set(x)
    zeros128 = jnp.zeros((NP, 128), F32)

    w1r = W1e.reshape(21, 26, 64)
    w1p = jnp.zeros((21, 32, 64), F32).at[:, :26, :].set(w1r).reshape(672, 64)
    be1 = jnp.zeros((32, 64), F32).at[:26, :].set(b1e.reshape(26, 64))
    waug1 = jnp.concatenate([w1p, be1], axis=0)
    root1p = jnp.zeros((32, 64), F32).at[:26, :].set(root1)

    w2p = W2e.reshape(21, 64, 30).reshape(1344, 30)
    waug2 = jnp.concatenate([w2p, b2e.reshape(64, 30)], axis=0)
    root2p = jnp.zeros((64, 32), F32).at[:, :30].set(root2)
    bias2p = jnp.zeros((1, 32), F32).at[0, :30].set(bias2)
    g2p = jnp.zeros((1, 32), F32).at[0, :30].set(ln2_g)
    b2p = jnp.zeros((1, 32), F32).at[0, :30].set(ln2_b)

    wihft = jnp.zeros((32, 512), F32)
    wihbt = jnp.zeros((32, 512), F32)
    xbf = jnp.zeros((1, 512), F32)
    xbb = jnp.zeros((1, 512), F32)
    wcat = jnp.zeros((128, 512), F32)
    bf_all = bih_f + bhh_f
    bb_all = bih_b + bhh_b
    for gi in range(4):
        fcol = slice(128 * gi, 128 * gi + 64)
        bcol = slice(128 * gi + 64, 128 * gi + 128)
        gsl = slice(64 * gi, 64 * gi + 64)
        wihft = wihft.at[:30, fcol].set(Wih_f.T[:, gsl])
        wihbt = wihbt.at[:30, bcol].set(Wih_b.T[:, gsl])
        xbf = xbf.at[0, fcol].set(bf_all[gsl])
        xbb = xbb.at[0, bcol].set(bb_all[gsl])
        wcat = wcat.at[0:64, fcol].set(Whh_f.T[:, gsl])
        wcat = wcat.at[64:128, bcol].set(Whh_b.T[:, gsl])
    fcwt = jnp.zeros((128, 8), F32).at[:, :2].set(fcW.T)
    fcbp = jnp.zeros((1, 8), F32).at[0, :2].set(fcb)

    sbr1 = jnp.kron(jnp.eye(21, 22, dtype=F32), jnp.ones((1, 32), F32))
    tbr1 = jnp.kron(jnp.ones((1, 22), F32), jnp.eye(32, dtype=F32))
    obr1 = jnp.zeros((1, 22 * 32), F32).at[0, 21 * 32:].set(1.0)
    sbr2 = jnp.kron(jnp.eye(21, 22, dtype=F32), jnp.ones((1, 64), F32))
    tbr2 = jnp.kron(jnp.ones((1, 22), F32), jnp.eye(64, dtype=F32))
    obr2 = jnp.zeros((1, 22 * 64), F32).at[0, 21 * 64:].set(1.0)

    hsrc1 = _sc_gather(x_p, src2d, 128)
    msg1 = _tc_msg(ea_p, hsrc1, waug1, sbr1, tbr1, obr1, 32, 64, True)
    sums1 = _sc_scatter_add(msg1, dst2d, zeros128, 128)
    h1, cnt8 = _tc_combine1(sums1[0], sums1[1], x_p, root1p,
                            bias1[None, :], ln1_g[None, :], ln1_b[None, :])

    hsrc2 = _sc_gather(h1, src2d, 128)
    msg2 = _tc_msg(ea_p, hsrc2, waug2, sbr2, tbr2, obr2, 64, 30, False)
    sums2 = _sc_scatter_add(msg2, dst2d, zeros128, 128)

    h2 = _tc_combine2(sums2[0][:, :32], sums2[1][:, :32], cnt8,
                      h1[:, :64], root2p, bias2p, g2p, b2p)
    prev = jnp.eye(512, dtype=F32)[::-1]
    out = _tc_lstm_head(h2, wihft, wihbt, xbf, xbb, wcat,
                        ln3_g[None, :], ln3_b[None, :], fcwt, fcbp, prev)
    return out[:N, :2]


# speedup vs baseline: 1.0149x; 1.0149x over previous
import functools

import jax
import jax.numpy as jnp
from jax import lax
from jax.experimental import pallas as pl
from jax.experimental.pallas import tpu as pltpu
from jax.experimental.pallas import tpu_sc as plsc

N = 10000
E = 160000
NP = 10240
EP = 163840
DUMMY = 10000

NWORK = 32
EPW = EP // NWORK
CHUNK = 128
NCHUNK = EPW // CHUNK

F32 = jnp.float32


_NBUF = 4


def _sc_gather(table, idx2d, d):
    mesh = plsc.VectorSubcoreMesh(core_axis_name="c", subcore_axis_name="s")

    @functools.partial(
        pl.kernel, mesh=mesh,
        out_type=jax.ShapeDtypeStruct((EP, d), F32),
        scratch_types=[
            pltpu.VMEM((NCHUNK, CHUNK), jnp.int32),
            [pltpu.VMEM((CHUNK, d), F32) for _ in range(_NBUF)],
            pltpu.SemaphoreType.DMA,
            pltpu.SemaphoreType.DMA,
        ],
    )
    def k(table_hbm, idx_hbm, out_hbm, idx_v, bufs, gsem, wsem):
        wid = lax.axis_index("s") * 2 + lax.axis_index("c")
        base = wid * EPW
        pltpu.sync_copy(idx_hbm.at[pl.ds(wid * NCHUNK, NCHUNK)], idx_v)

        def body(j, carry):
            pltpu.async_copy(table_hbm.at[idx_v.at[j]], bufs[0], gsem).wait()
            pltpu.sync_copy(bufs[0],
                            out_hbm.at[pl.ds(base + j * CHUNK, CHUNK)])
            return carry

        lax.fori_loop(0, NCHUNK, body, 0)

    return k(table, idx2d)


def _sc_scatter_add(msg, dst2d, zeros_np, d):
    mesh = plsc.VectorSubcoreMesh(core_axis_name="c", subcore_axis_name="s")
    rpt = NP // 16

    nb = 2

    @functools.partial(
        pl.kernel, mesh=mesh,
        out_type=jax.ShapeDtypeStruct((2, NP, d), F32),
        scratch_types=[
            pltpu.VMEM((NCHUNK, CHUNK), jnp.int32),
            [pltpu.VMEM((CHUNK, d), F32) for _ in range(nb)],
            pltpu.VMEM_SHARED((NP, d), F32),
            pltpu.SemaphoreType.DMA,
            pltpu.SemaphoreType.DMA,
        ],
    )
    def k(msg_hbm, dst_hbm, zeros_hbm, out_hbm, idx_v, bufs, acc_sh,
          msem, ssem):
        cid = lax.axis_index("c")
        sid = lax.axis_index("s")
        wid = sid * 2 + cid
        base = wid * EPW
        pltpu.sync_copy(zeros_hbm.at[pl.ds(sid * rpt, rpt)],
                        acc_sh.at[pl.ds(sid * rpt, rpt)])
        pltpu.sync_copy(dst_hbm.at[pl.ds(wid * NCHUNK, NCHUNK)], idx_v)
        plsc.subcore_barrier()

        def body(j, carry):
            pltpu.sync_copy(msg_hbm.at[pl.ds(base + j * CHUNK, CHUNK)],
                            bufs[0])
            pltpu.sync_copy(bufs[0], acc_sh.at[idx_v.at[j]], add=True)
            return carry

        lax.fori_loop(0, NCHUNK, body, 0)
        plsc.subcore_barrier()
        pltpu.sync_copy(acc_sh.at[pl.ds(sid * rpt, rpt)],
                        out_hbm.at[cid, pl.ds(sid * rpt, rpt)])

    return k(msg, dst2d, zeros_np)


_ET = 1024


def _tc_msg(ea, hsrc, waug, sbr, tbr, obr, cw, dout, ones_col):
    grid = EP // _ET

    def body(ea_ref, hs_ref, w_ref, s_ref, t_ref, o_ref, out_ref):
        eat = ea_ref[...]
        hs = hs_ref[...][:, :cw]
        eab = jnp.dot(eat, s_ref[...], preferred_element_type=F32) + o_ref[...]
        htl = jnp.dot(hs, t_ref[...], preferred_element_type=F32)
        g = eab * htl
        msg = jnp.dot(g, w_ref[...], preferred_element_type=F32)
        lane = lax.broadcasted_iota(jnp.int32, (_ET, 128 - dout), 1)
        tailv = 1.0 if ones_col else 0.0
        tail = jnp.where(lane == 0, tailv, 0.0).astype(F32)
        out_ref[...] = jnp.concatenate([msg, tail], axis=1)

    return pl.pallas_call(
        body,
        grid=(grid,),
        in_specs=[
            pl.BlockSpec((_ET, 21), lambda i: (i, 0)),
            pl.BlockSpec((_ET, 128), lambda i: (i, 0)),
            pl.BlockSpec((22 * cw, dout), lambda i: (0, 0)),
            pl.BlockSpec((21, 22 * cw), lambda i: (0, 0)),
            pl.BlockSpec((cw, 22 * cw), lambda i: (0, 0)),
            pl.BlockSpec((1, 22 * cw), lambda i: (0, 0)),
        ],
        out_specs=pl.BlockSpec((_ET, 128), lambda i: (i, 0)),
        out_shape=jax.ShapeDtypeStruct((EP, 128), F32),
    )(ea, hsrc, waug, sbr, tbr, obr)


_NT = 512


def _tc_combine1(s0, s1, xp, rootp, biasp, lng, lnb):
    grid = NP // _NT

    def body(s0_ref, s1_ref, x_ref, r_ref, b_ref, g_ref, be_ref,
             h_ref, c_ref):
        s = s0_ref[...] + s1_ref[...]
        cnt = s[:, 64:65]
        agg = s[:, :64] / jnp.maximum(cnt, 1.0)
        pre = agg + jnp.dot(x_ref[...][:, :32], r_ref[...],
                            preferred_element_type=F32) + b_ref[...]
        m = jnp.mean(pre, axis=-1, keepdims=True)
        v = jnp.mean((pre - m) ** 2, axis=-1, keepdims=True)
        h = (pre - m) / jnp.sqrt(v + 1e-5) * g_ref[...] + be_ref[...]
        h = jnp.where(h >= 0, h, 0.01 * h)
        h_ref[...] = jnp.concatenate([h, jnp.zeros((_NT, 64), F32)], axis=1)
        c_ref[...] = jnp.broadcast_to(cnt, (_NT, 8))

    return pl.pallas_call(
        body,
        grid=(grid,),
        in_specs=[
            pl.BlockSpec((_NT, 128), lambda i: (i, 0)),
            pl.BlockSpec((_NT, 128), lambda i: (i, 0)),
            pl.BlockSpec((_NT, 128), lambda i: (i, 0)),
            pl.BlockSpec((32, 64), lambda i: (0, 0)),
            pl.BlockSpec((1, 64), lambda i: (0, 0)),
            pl.BlockSpec((1, 64), lambda i: (0, 0)),
            pl.BlockSpec((1, 64), lambda i: (0, 0)),
        ],
        out_specs=[
            pl.BlockSpec((_NT, 128), lambda i: (i, 0)),
            pl.BlockSpec((_NT, 8), lambda i: (i, 0)),
        ],
        out_shape=[
            jax.ShapeDtypeStruct((NP, 128), F32),
            jax.ShapeDtypeStruct((NP, 8), F32),
        ],
    )(s0, s1, xp, rootp, biasp, lng, lnb)


def _tc_combine2(s0, s1, cnt8, h1s, root2p, bias2p, g2p, b2p):
    grid = NP // _NT

    def body(s0_ref, s1_ref, c_ref, h1_ref, r2_ref, b2_ref, g2_ref, be2_ref,
             h2_ref):
        s = s0_ref[...] + s1_ref[...]
        cnt = c_ref[...][:, 0:1]
        agg = s / jnp.maximum(cnt, 1.0)
        pre = agg + jnp.dot(h1_ref[...], r2_ref[...],
                            preferred_element_type=F32) + b2_ref[...]
        m = jnp.sum(pre, axis=-1, keepdims=True) * (1.0 / 30.0)
        d = pre[:, :30] - m
        v = jnp.sum(d * d, axis=-1, keepdims=True) * (1.0 / 30.0)
        h2 = (pre - m) / jnp.sqrt(v + 1e-5) * g2_ref[...] + be2_ref[...]
        h2_ref[...] = jnp.where(h2 >= 0, h2, 0.01 * h2)

    return pl.pallas_call(
        body,
        grid=(grid,),
        in_specs=[
            pl.BlockSpec((_NT, 32), lambda i: (i, 0)),
            pl.BlockSpec((_NT, 32), lambda i: (i, 0)),
            pl.BlockSpec((_NT, 8), lambda i: (i, 0)),
            pl.BlockSpec((_NT, 64), lambda i: (i, 0)),
            pl.BlockSpec((64, 32), lambda i: (0, 0)),
            pl.BlockSpec((1, 32), lambda i: (0, 0)),
            pl.BlockSpec((1, 32), lambda i: (0, 0)),
            pl.BlockSpec((1, 32), lambda i: (0, 0)),
        ],
        out_specs=pl.BlockSpec((_NT, 32), lambda i: (i, 0)),
        out_shape=jax.ShapeDtypeStruct((NP, 32), F32),
    )(s0, s1, cnt8, h1s, root2p, bias2p, g2p, b2p)


def _tc_lstm_head(h2, wihft, wihbt, xbf, xbb, wcat, g3, b3, fcwt, fcbp, prev):
    ntiles = NP // _NT

    def body(h2_ref, wf_ref, wb_ref, xbf_ref, xbb_ref, wc_ref, g3_ref,
             b3_ref, fw_ref, fb_ref, pr_ref, out_ref, xsum_scr, h_scr):
        def stage1b(j, carry):
            src = h2_ref[pl.ds(j * _NT, _NT), :]
            xb = jnp.dot(src, wb_ref[...],
                         preferred_element_type=F32) + xbb_ref[...]
            rev = jnp.dot(pr_ref[...], xb, preferred_element_type=F32,
                          precision=lax.Precision.HIGHEST)
            xsum_scr[pl.ds((N - 512) - _NT * j, _NT), :] = rev
            return carry

        lax.fori_loop(0, ntiles - 1, stage1b, 0)
        srcl = h2_ref[pl.ds((ntiles - 1) * _NT, _NT), :]
        xbl = jnp.dot(srcl, wb_ref[...],
                      preferred_element_type=F32) + xbb_ref[...]
        revl = jnp.dot(pr_ref[...], xbl, preferred_element_type=F32,
                       precision=lax.Precision.HIGHEST)
        xsum_scr[pl.ds(0, N - (ntiles - 1) * _NT), :] = (
            revl[_NT - (N - (ntiles - 1) * _NT):, :])

        def stage1f(i, carry):
            sl = pl.ds(i * _NT, _NT)
            xsum_scr[sl, :] = (xsum_scr[sl, :]
                               + jnp.dot(h2_ref[sl, :], wf_ref[...],
                                         preferred_element_type=F32)
                               + xbf_ref[...])
            return carry

        lax.fori_loop(0, ntiles, stage1f, 0)

        wc = wc_ref[...]

        def step(t, carry):
            h, c = carry
            rt = (N - 1) - t
            g = jnp.dot(h, wc, preferred_element_type=F32)
            gate = g + xsum_scr[pl.ds(t, 1), :]
            i_a = jax.nn.sigmoid(gate[:, 0:128])
            f_a = jax.nn.sigmoid(gate[:, 128:256])
            n_a = jnp.tanh(gate[:, 256:384])
            o_a = jax.nn.sigmoid(gate[:, 384:512])
            c = f_a * c + i_a * n_a
            h = o_a * jnp.tanh(c)
            h_scr[pl.ds(t, 1), 0:64] = h[:, 0:64]
            h_scr[pl.ds(rt, 1), 64:128] = h[:, 64:128]
            return h, c

        z = jnp.zeros((1, 128), F32)
        lax.fori_loop(0, N, step, (z, z), unroll=4)

        def head(i, carry):
            sl = pl.ds(i * _NT, _NT)
            hc = h_scr[sl, :]
            m = jnp.mean(hc, axis=-1, keepdims=True)
            v = jnp.mean((hc - m) ** 2, axis=-1, keepdims=True)
            hn = (hc - m) / jnp.sqrt(v + 1e-5) * g3_ref[...] + b3_ref[...]
            out_ref[sl, :] = jnp.dot(hn, fw_ref[...],
                                     preferred_element_type=F32) + fb_ref[...]
            return carry

        lax.fori_loop(0, ntiles, head, 0)

    return pl.pallas_call(
        body,
        out_shape=jax.ShapeDtypeStruct((NP, 8), F32),
        scratch_shapes=[
            pltpu.VMEM((NP, 512), F32),
            pltpu.VMEM((NP, 128), F32),
        ],
    )(h2, wihft, wihbt, xbf, xbb, wcat, g3, b3, fcwt, fcbp, prev)


def kernel(x, edge_attr, edge_index, W1e, b1e, root1, bias1, ln1_g, ln1_b,
           W2e, b2e, root2, bias2, ln2_g, ln2_b,
           Wih_f, Whh_f, bih_f, bhh_f, Wih_b, Whh_b, bih_b, bhh_b,
           ln3_g, ln3_b, fcW, fcb):
    src = edge_index[0]
    dst = edge_index[1]
    src_p = jnp.concatenate([src, jnp.zeros((EP - E,), jnp.int32)])
    dst_p = jnp.concatenate([dst, jnp.full((EP - E,), DUMMY, jnp.int32)])
    src2d = src_p.reshape(EP // CHUNK, CHUNK)
    dst2d = dst_p.reshape(EP // CHUNK, CHUNK)
    ea_p = jnp.concatenate([edge_attr, jnp.zeros((EP - E, 21), F32)], axis=0)
    x_p = jnp.zeros((NP, 128), F32).at[:N, :26].set(x)
    zeros128 = jnp.zeros((NP, 128), F32)

    w1r = W1e.reshape(21, 26, 64)
    w1p = jnp.zeros((21, 32, 64), F32).at[:, :26, :].set(w1r).reshape(672, 64)
    be1 = jnp.zeros((32, 64), F32).at[:26, :].set(b1e.reshape(26, 64))
    waug1 = jnp.concatenate([w1p, be1], axis=0)
    root1p = jnp.zeros((32, 64), F32).at[:26, :].set(root1)

    w2p = W2e.reshape(21, 64, 30).reshape(1344, 30)
    waug2 = jnp.concatenate([w2p, b2e.reshape(64, 30)], axis=0)
    root2p = jnp.zeros((64, 32), F32).at[:, :30].set(root2)
    bias2p = jnp.zeros((1, 32), F32).at[0, :30].set(bias2)
    g2p = jnp.zeros((1, 32), F32).at[0, :30].set(ln2_g)
    b2p = jnp.zeros((1, 32), F32).at[0, :30].set(ln2_b)

    wihft = jnp.zeros((32, 512), F32)
    wihbt = jnp.zeros((32, 512), F32)
    xbf = jnp.zeros((1, 512), F32)
    xbb = jnp.zeros((1, 512), F32)
    wcat = jnp.zeros((128, 512), F32)
    bf_all = bih_f + bhh_f
    bb_all = bih_b + bhh_b
    for gi in range(4):
        fcol = slice(128 * gi, 128 * gi + 64)
        bcol = slice(128 * gi + 64, 128 * gi + 128)
        gsl = slice(64 * gi, 64 * gi + 64)
        wihft = wihft.at[:30, fcol].set(Wih_f.T[:, gsl])
        wihbt = wihbt.at[:30, bcol].set(Wih_b.T[:, gsl])
        xbf = xbf.at[0, fcol].set(bf_all[gsl])
        xbb = xbb.at[0, bcol].set(bb_all[gsl])
        wcat = wcat.at[0:64, fcol].set(Whh_f.T[:, gsl])
        wcat = wcat.at[64:128, bcol].set(Whh_b.T[:, gsl])
    fcwt = jnp.zeros((128, 8), F32).at[:, :2].set(fcW.T)
    fcbp = jnp.zeros((1, 8), F32).at[0, :2].set(fcb)

    sbr1 = jnp.kron(jnp.eye(21, 22, dtype=F32), jnp.ones((1, 32), F32))
    tbr1 = jnp.kron(jnp.ones((1, 22), F32), jnp.eye(32, dtype=F32))
    obr1 = jnp.zeros((1, 22 * 32), F32).at[0, 21 * 32:].set(1.0)
    sbr2 = jnp.kron(jnp.eye(21, 22, dtype=F32), jnp.ones((1, 64), F32))
    tbr2 = jnp.kron(jnp.ones((1, 22), F32), jnp.eye(64, dtype=F32))
    obr2 = jnp.zeros((1, 22 * 64), F32).at[0, 21 * 64:].set(1.0)

    hsrc1 = _sc_gather(x_p, src2d, 128)
    msg1 = _tc_msg(ea_p, hsrc1, waug1, sbr1, tbr1, obr1, 32, 64, True)
    sums1 = _sc_scatter_add(msg1, dst2d, zeros128, 128)
    h1, cnt8 = _tc_combine1(sums1[0], sums1[1], x_p, root1p,
                            bias1[None, :], ln1_g[None, :], ln1_b[None, :])

    hsrc2 = _sc_gather(h1, src2d, 128)
    msg2 = _tc_msg(ea_p, hsrc2, waug2, sbr2, tbr2, obr2, 64, 30, False)
    sums2 = _sc_scatter_add(msg2, dst2d, zeros128, 128)

    h2 = _tc_combine2(sums2[0][:, :32], sums2[1][:, :32], cnt8,
                      h1[:, :64], root2p, bias2p, g2p, b2p)
    prev = jnp.eye(512, dtype=F32)[::-1]
    out = _tc_lstm_head(h2, wihft, wihbt, xbf, xbb, wcat,
                        ln3_g[None, :], ln3_b[None, :], fcwt, fcbp, prev)
    return out[:N, :2]


# speedup vs baseline: 1.0275x; 1.0124x over previous
import functools

import jax
import jax.numpy as jnp
from jax import lax
from jax.experimental import pallas as pl
from jax.experimental.pallas import tpu as pltpu
from jax.experimental.pallas import tpu_sc as plsc

N = 10000
E = 160000
NP = 10240
EP = 163840
DUMMY = 10000

NWORK = 32
EPW = EP // NWORK
CHUNK = 128
NCHUNK = EPW // CHUNK

F32 = jnp.float32


_NBUF = 4


def _sc_gather(table, idx2d, d):
    mesh = plsc.VectorSubcoreMesh(core_axis_name="c", subcore_axis_name="s")

    @functools.partial(
        pl.kernel, mesh=mesh,
        out_type=jax.ShapeDtypeStruct((EP, d), F32),
        scratch_types=[
            pltpu.VMEM((NCHUNK, CHUNK), jnp.int32),
            [pltpu.VMEM((CHUNK, d), F32) for _ in range(_NBUF)],
            pltpu.SemaphoreType.DMA,
            pltpu.SemaphoreType.DMA,
        ],
    )
    def k(table_hbm, idx_hbm, out_hbm, idx_v, bufs, gsem, wsem):
        wid = lax.axis_index("s") * 2 + lax.axis_index("c")
        base = wid * EPW
        pltpu.sync_copy(idx_hbm.at[pl.ds(wid * NCHUNK, NCHUNK)], idx_v)

        def group(g, carry):
            gds = [pltpu.async_copy(table_hbm.at[idx_v.at[g * _NBUF + b]],
                                    bufs[b], gsem)
                   for b in range(_NBUF)]
            for dsc in gds:
                dsc.wait()
            wds = [pltpu.async_copy(
                bufs[b],
                out_hbm.at[pl.ds(base + (g * _NBUF + b) * CHUNK, CHUNK)],
                wsem)
                for b in range(_NBUF)]
            for dsc in wds:
                dsc.wait()
            return carry

        lax.fori_loop(0, NCHUNK // _NBUF, group, 0)

    return k(table, idx2d)


def _sc_scatter_add(msg, dst2d, zeros_np, d):
    mesh = plsc.VectorSubcoreMesh(core_axis_name="c", subcore_axis_name="s")
    rpt = NP // 16

    nb = 2

    @functools.partial(
        pl.kernel, mesh=mesh,
        out_type=jax.ShapeDtypeStruct((2, NP, d), F32),
        scratch_types=[
            pltpu.VMEM((NCHUNK, CHUNK), jnp.int32),
            [pltpu.VMEM((CHUNK, d), F32) for _ in range(nb)],
            pltpu.VMEM_SHARED((NP, d), F32),
            pltpu.SemaphoreType.DMA,
            pltpu.SemaphoreType.DMA,
        ],
    )
    def k(msg_hbm, dst_hbm, zeros_hbm, out_hbm, idx_v, bufs, acc_sh,
          msem, ssem):
        cid = lax.axis_index("c")
        sid = lax.axis_index("s")
        wid = sid * 2 + cid
        base = wid * EPW
        pltpu.sync_copy(zeros_hbm.at[pl.ds(sid * rpt, rpt)],
                        acc_sh.at[pl.ds(sid * rpt, rpt)])
        pltpu.sync_copy(dst_hbm.at[pl.ds(wid * NCHUNK, NCHUNK)], idx_v)
        plsc.subcore_barrier()

        def body(j, carry):
            pltpu.sync_copy(msg_hbm.at[pl.ds(base + j * CHUNK, CHUNK)],
                            bufs[0])
            pltpu.sync_copy(bufs[0], acc_sh.at[idx_v.at[j]], add=True)
            return carry

        lax.fori_loop(0, NCHUNK, body, 0)
        plsc.subcore_barrier()
        pltpu.sync_copy(acc_sh.at[pl.ds(sid * rpt, rpt)],
                        out_hbm.at[cid, pl.ds(sid * rpt, rpt)])

    return k(msg, dst2d, zeros_np)


_ET = 1024


def _tc_msg(ea, hsrc, waug, sbr, tbr, obr, cw, dout, ones_col):
    grid = EP // _ET

    def body(ea_ref, hs_ref, w_ref, s_ref, t_ref, o_ref, out_ref):
        eat = ea_ref[...]
        hs = hs_ref[...][:, :cw]
        eab = jnp.dot(eat, s_ref[...], preferred_element_type=F32) + o_ref[...]
        htl = jnp.dot(hs, t_ref[...], preferred_element_type=F32)
        g = eab * htl
        msg = jnp.dot(g, w_ref[...], preferred_element_type=F32)
        lane = lax.broadcasted_iota(jnp.int32, (_ET, 128 - dout), 1)
        tailv = 1.0 if ones_col else 0.0
        tail = jnp.where(lane == 0, tailv, 0.0).astype(F32)
        out_ref[...] = jnp.concatenate([msg, tail], axis=1)

    return pl.pallas_call(
        body,
        grid=(grid,),
        in_specs=[
            pl.BlockSpec((_ET, 21), lambda i: (i, 0)),
            pl.BlockSpec((_ET, 128), lambda i: (i, 0)),
            pl.BlockSpec((22 * cw, dout), lambda i: (0, 0)),
            pl.BlockSpec((21, 22 * cw), lambda i: (0, 0)),
            pl.BlockSpec((cw, 22 * cw), lambda i: (0, 0)),
            pl.BlockSpec((1, 22 * cw), lambda i: (0, 0)),
        ],
        out_specs=pl.BlockSpec((_ET, 128), lambda i: (i, 0)),
        out_shape=jax.ShapeDtypeStruct((EP, 128), F32),
    )(ea, hsrc, waug, sbr, tbr, obr)


_NT = 512


def _tc_combine1(s0, s1, xp, rootp, biasp, lng, lnb):
    grid = NP // _NT

    def body(s0_ref, s1_ref, x_ref, r_ref, b_ref, g_ref, be_ref,
             h_ref, c_ref):
        s = s0_ref[...] + s1_ref[...]
        cnt = s[:, 64:65]
        agg = s[:, :64] / jnp.maximum(cnt, 1.0)
        pre = agg + jnp.dot(x_ref[...][:, :32], r_ref[...],
                            preferred_element_type=F32) + b_ref[...]
        m = jnp.mean(pre, axis=-1, keepdims=True)
        v = jnp.mean((pre - m) ** 2, axis=-1, keepdims=True)
        h = (pre - m) / jnp.sqrt(v + 1e-5) * g_ref[...] + be_ref[...]
        h = jnp.where(h >= 0, h, 0.01 * h)
        h_ref[...] = jnp.concatenate([h, jnp.zeros((_NT, 64), F32)], axis=1)
        c_ref[...] = jnp.broadcast_to(cnt, (_NT, 8))

    return pl.pallas_call(
        body,
        grid=(grid,),
        in_specs=[
            pl.BlockSpec((_NT, 128), lambda i: (i, 0)),
            pl.BlockSpec((_NT, 128), lambda i: (i, 0)),
            pl.BlockSpec((_NT, 128), lambda i: (i, 0)),
            pl.BlockSpec((32, 64), lambda i: (0, 0)),
            pl.BlockSpec((1, 64), lambda i: (0, 0)),
            pl.BlockSpec((1, 64), lambda i: (0, 0)),
            pl.BlockSpec((1, 64), lambda i: (0, 0)),
        ],
        out_specs=[
            pl.BlockSpec((_NT, 128), lambda i: (i, 0)),
            pl.BlockSpec((_NT, 8), lambda i: (i, 0)),
        ],
        out_shape=[
            jax.ShapeDtypeStruct((NP, 128), F32),
            jax.ShapeDtypeStruct((NP, 8), F32),
        ],
    )(s0, s1, xp, rootp, biasp, lng, lnb)


def _tc_combine2(s0, s1, cnt8, h1s, root2p, bias2p, g2p, b2p):
    grid = NP // _NT

    def body(s0_ref, s1_ref, c_ref, h1_ref, r2_ref, b2_ref, g2_ref, be2_ref,
             h2_ref):
        s = s0_ref[...] + s1_ref[...]
        cnt = c_ref[...][:, 0:1]
        agg = s / jnp.maximum(cnt, 1.0)
        pre = agg + jnp.dot(h1_ref[...], r2_ref[...],
                            preferred_element_type=F32) + b2_ref[...]
        m = jnp.sum(pre, axis=-1, keepdims=True) * (1.0 / 30.0)
        d = pre[:, :30] - m
        v = jnp.sum(d * d, axis=-1, keepdims=True) * (1.0 / 30.0)
        h2 = (pre - m) / jnp.sqrt(v + 1e-5) * g2_ref[...] + be2_ref[...]
        h2_ref[...] = jnp.where(h2 >= 0, h2, 0.01 * h2)

    return pl.pallas_call(
        body,
        grid=(grid,),
        in_specs=[
            pl.BlockSpec((_NT, 32), lambda i: (i, 0)),
            pl.BlockSpec((_NT, 32), lambda i: (i, 0)),
            pl.BlockSpec((_NT, 8), lambda i: (i, 0)),
            pl.BlockSpec((_NT, 64), lambda i: (i, 0)),
            pl.BlockSpec((64, 32), lambda i: (0, 0)),
            pl.BlockSpec((1, 32), lambda i: (0, 0)),
            pl.BlockSpec((1, 32), lambda i: (0, 0)),
            pl.BlockSpec((1, 32), lambda i: (0, 0)),
        ],
        out_specs=pl.BlockSpec((_NT, 32), lambda i: (i, 0)),
        out_shape=jax.ShapeDtypeStruct((NP, 32), F32),
    )(s0, s1, cnt8, h1s, root2p, bias2p, g2p, b2p)


def _tc_lstm_head(h2, wihft, wihbt, xbf, xbb, wcat, g3, b3, fcwt, fcbp, prev):
    ntiles = NP // _NT

    def body(h2_ref, wf_ref, wb_ref, xbf_ref, xbb_ref, wc_ref, g3_ref,
             b3_ref, fw_ref, fb_ref, pr_ref, out_ref, xsum_scr, h_scr):
        def stage1b(j, carry):
            src = h2_ref[pl.ds(j * _NT, _NT), :]
            xb = jnp.dot(src, wb_ref[...],
                         preferred_element_type=F32) + xbb_ref[...]
            rev = jnp.dot(pr_ref[...], xb, preferred_element_type=F32,
                          precision=lax.Precision.HIGHEST)
            xsum_scr[pl.ds((N - 512) - _NT * j, _NT), :] = rev
            return carry

        lax.fori_loop(0, ntiles - 1, stage1b, 0)
        srcl = h2_ref[pl.ds((ntiles - 1) * _NT, _NT), :]
        xbl = jnp.dot(srcl, wb_ref[...],
                      preferred_element_type=F32) + xbb_ref[...]
        revl = jnp.dot(pr_ref[...], xbl, preferred_element_type=F32,
                       precision=lax.Precision.HIGHEST)
        xsum_scr[pl.ds(0, N - (ntiles - 1) * _NT), :] = (
            revl[_NT - (N - (ntiles - 1) * _NT):, :])

        def stage1f(i, carry):
            sl = pl.ds(i * _NT, _NT)
            xsum_scr[sl, :] = (xsum_scr[sl, :]
                               + jnp.dot(h2_ref[sl, :], wf_ref[...],
                                         preferred_element_type=F32)
                               + xbf_ref[...])
            return carry

        lax.fori_loop(0, ntiles, stage1f, 0)

        wc = wc_ref[...]

        def step(t, carry):
            h, c = carry
            rt = (N - 1) - t
            g = jnp.dot(h, wc, preferred_element_type=F32)
            gate = g + xsum_scr[pl.ds(t, 1), :]
            i_a = jax.nn.sigmoid(gate[:, 0:128])
            f_a = jax.nn.sigmoid(gate[:, 128:256])
            n_a = jnp.tanh(gate[:, 256:384])
            o_a = jax.nn.sigmoid(gate[:, 384:512])
            c = f_a * c + i_a * n_a
            h = o_a * jnp.tanh(c)
            h_scr[pl.ds(t, 1), 0:64] = h[:, 0:64]
            h_scr[pl.ds(rt, 1), 64:128] = h[:, 64:128]
            return h, c

        z = jnp.zeros((1, 128), F32)
        lax.fori_loop(0, N, step, (z, z), unroll=4)

        def head(i, carry):
            sl = pl.ds(i * _NT, _NT)
            hc = h_scr[sl, :]
            m = jnp.mean(hc, axis=-1, keepdims=True)
            v = jnp.mean((hc - m) ** 2, axis=-1, keepdims=True)
            hn = (hc - m) / jnp.sqrt(v + 1e-5) * g3_ref[...] + b3_ref[...]
            out_ref[sl, :] = jnp.dot(hn, fw_ref[...],
                                     preferred_element_type=F32) + fb_ref[...]
            return carry

        lax.fori_loop(0, ntiles, head, 0)

    return pl.pallas_call(
        body,
        out_shape=jax.ShapeDtypeStruct((NP, 8), F32),
        scratch_shapes=[
            pltpu.VMEM((NP, 512), F32),
            pltpu.VMEM((NP, 128), F32),
        ],
    )(h2, wihft, wihbt, xbf, xbb, wcat, g3, b3, fcwt, fcbp, prev)


def kernel(x, edge_attr, edge_index, W1e, b1e, root1, bias1, ln1_g, ln1_b,
           W2e, b2e, root2, bias2, ln2_g, ln2_b,
           Wih_f, Whh_f, bih_f, bhh_f, Wih_b, Whh_b, bih_b, bhh_b,
           ln3_g, ln3_b, fcW, fcb):
    src = edge_index[0]
    dst = edge_index[1]
    src_p = jnp.concatenate([src, jnp.zeros((EP - E,), jnp.int32)])
    dst_p = jnp.concatenate([dst, jnp.full((EP - E,), DUMMY, jnp.int32)])
    src2d = src_p.reshape(EP // CHUNK, CHUNK)
    dst2d = dst_p.reshape(EP // CHUNK, CHUNK)
    ea_p = jnp.concatenate([edge_attr, jnp.zeros((EP - E, 21), F32)], axis=0)
    x_p = jnp.zeros((NP, 128), F32).at[:N, :26].set(x)
    zeros128 = jnp.zeros((NP, 128), F32)

    w1r = W1e.reshape(21, 26, 64)
    w1p = jnp.zeros((21, 32, 64), F32).at[:, :26, :].set(w1r).reshape(672, 64)
    be1 = jnp.zeros((32, 64), F32).at[:26, :].set(b1e.reshape(26, 64))
    waug1 = jnp.concatenate([w1p, be1], axis=0)
    root1p = jnp.zeros((32, 64), F32).at[:26, :].set(root1)

    w2p = W2e.reshape(21, 64, 30).reshape(1344, 30)
    waug2 = jnp.concatenate([w2p, b2e.reshape(64, 30)], axis=0)
    root2p = jnp.zeros((64, 32), F32).at[:, :30].set(root2)
    bias2p = jnp.zeros((1, 32), F32).at[0, :30].set(bias2)
    g2p = jnp.zeros((1, 32), F32).at[0, :30].set(ln2_g)
    b2p = jnp.zeros((1, 32), F32).at[0, :30].set(ln2_b)

    wihft = jnp.zeros((32, 512), F32)
    wihbt = jnp.zeros((32, 512), F32)
    xbf = jnp.zeros((1, 512), F32)
    xbb = jnp.zeros((1, 512), F32)
    wcat = jnp.zeros((128, 512), F32)
    bf_all = bih_f + bhh_f
    bb_all = bih_b + bhh_b
    for gi in range(4):
        fcol = slice(128 * gi, 128 * gi + 64)
        bcol = slice(128 * gi + 64, 128 * gi + 128)
        gsl = slice(64 * gi, 64 * gi + 64)
        wihft = wihft.at[:30, fcol].set(Wih_f.T[:, gsl])
        wihbt = wihbt.at[:30, bcol].set(Wih_b.T[:, gsl])
        xbf = xbf.at[0, fcol].set(bf_all[gsl])
        xbb = xbb.at[0, bcol].set(bb_all[gsl])
        wcat = wcat.at[0:64, fcol].set(Whh_f.T[:, gsl])
        wcat = wcat.at[64:128, bcol].set(Whh_b.T[:, gsl])
    fcwt = jnp.zeros((128, 8), F32).at[:, :2].set(fcW.T)
    fcbp = jnp.zeros((1, 8), F32).at[0, :2].set(fcb)

    sbr1 = jnp.kron(jnp.eye(21, 22, dtype=F32), jnp.ones((1, 32), F32))
    tbr1 = jnp.kron(jnp.ones((1, 22), F32), jnp.eye(32, dtype=F32))
    obr1 = jnp.zeros((1, 22 * 32), F32).at[0, 21 * 32:].set(1.0)
    sbr2 = jnp.kron(jnp.eye(21, 22, dtype=F32), jnp.ones((1, 64), F32))
    tbr2 = jnp.kron(jnp.ones((1, 22), F32), jnp.eye(64, dtype=F32))
    obr2 = jnp.zeros((1, 22 * 64), F32).at[0, 21 * 64:].set(1.0)

    hsrc1 = _sc_gather(x_p, src2d, 128)
    msg1 = _tc_msg(ea_p, hsrc1, waug1, sbr1, tbr1, obr1, 32, 64, True)
    sums1 = _sc_scatter_add(msg1, dst2d, zeros128, 128)
    h1, cnt8 = _tc_combine1(sums1[0], sums1[1], x_p, root1p,
                            bias1[None, :], ln1_g[None, :], ln1_b[None, :])

    hsrc2 = _sc_gather(h1, src2d, 128)
    msg2 = _tc_msg(ea_p, hsrc2, waug2, sbr2, tbr2, obr2, 64, 30, False)
    sums2 = _sc_scatter_add(msg2, dst2d, zeros128, 128)

    h2 = _tc_combine2(sums2[0][:, :32], sums2[1][:, :32], cnt8,
                      h1[:, :64], root2p, bias2p, g2p, b2p)
    prev = jnp.eye(512, dtype=F32)[::-1]
    out = _tc_lstm_head(h2, wihft, wihbt, xbf, xbb, wcat,
                        ln3_g[None, :], ln3_b[None, :], fcwt, fcbp, prev)
    return out[:N, :2]
